# Initial kernel scaffold; baseline (speedup 1.0000x reference)
import jax
import jax.numpy as jnp
from jax.experimental import pallas as pl

WINDOW = 10


def kernel(data, action_table, rew, env_id, obs_next_idx, length):
    B = env_id.shape[0]
    a = action_table[obs_next_idx] * rew[:, None]
    j = jnp.arange(B, dtype=jnp.int32)
    winbuf = jnp.zeros((B,), jnp.int32).at[env_id].max(j)
    win = winbuf[env_id]
    vals = a[win]
    start = length - WINDOW
    wsum = jax.lax.dynamic_slice_in_dim(data, start, WINDOW - 1, axis=0)[:, env_id].sum(axis=0)
    return (wsum + vals) / jnp.float32(WINDOW)


# speedup vs baseline: 2.0209x; 2.0209x over previous
import functools

import jax
import jax.numpy as jnp
from jax import lax
from jax.experimental import pallas as pl
from jax.experimental.pallas import tpu as pltpu
from jax.experimental.pallas import tpu_sc as plsc

WINDOW = 10
NC = 2
NS = 16
L = 16


def _make_kernel(B, D, V, T):
    NW = NC * NS
    bw = B // NW
    nck = bw // L
    ncf = B // L
    nacc = WINDOW - 1
    mesh = plsc.VectorSubcoreMesh(core_axis_name="c", subcore_axis_name="s")

    @functools.partial(
        pl.kernel,
        mesh=mesh,
        out_type=jax.ShapeDtypeStruct((B, D), jnp.float32),
        compiler_params=pltpu.CompilerParams(
            needs_layout_passes=False, use_tc_tiling_on_sc=False),
        scratch_types=[
            pltpu.VMEM((B,), jnp.int32),
            pltpu.VMEM((B,), jnp.int32),
            pltpu.VMEM((B,), jnp.float32),
            pltpu.VMEM((B,), jnp.int32),
            pltpu.VMEM((L,), jnp.int32),
            pltpu.VMEM((nacc, bw), jnp.int32),
            pltpu.VMEM((bw,), jnp.int32),
            pltpu.VMEM((bw,), jnp.float32),
            pltpu.VMEM((nacc, bw, D), jnp.float32),
            pltpu.VMEM((bw, D), jnp.float32),
            pltpu.VMEM((bw, D), jnp.float32),
            pltpu.SemaphoreType.DMA,
            pltpu.SemaphoreType.DMA,
        ],
    )
    def k(data_hbm, atab_hbm, rew_hbm, env_hbm, obs_hbm, off_hbm, out_hbm,
          env_full, obs_full, rew_full, winbuf, off_v, idx2, aidx, rww,
          dbuf, abuf, obuf, sem_d, sem_a):
        wid = lax.axis_index("s") * NC + lax.axis_index("c")
        base = wid * bw

        pltpu.sync_copy(env_hbm, env_full)
        pltpu.sync_copy(obs_hbm, obs_full)
        pltpu.sync_copy(rew_hbm, rew_full)
        pltpu.sync_copy(off_hbm, off_v)

        ov = off_v[...]
        for t in range(nacc):
            offt = jnp.take(ov, jnp.full((L,), t, jnp.int32), mode="wrap")
            for c in range(nck):
                ev = env_full[pl.ds(base + c * L, L)]
                idx2[t, pl.ds(c * L, L)] = ev + offt

        dcopies = []
        for t in range(nacc):
            cp = pltpu.make_async_copy(data_hbm.at[idx2.at[t]], dbuf.at[t], sem_d)
            cp.start()
            dcopies.append(cp)

        lane = lax.iota(jnp.int32, L)

        def win_body(c, _):
            jv = lane + c * L
            ev = env_full[pl.ds(c * L, L)]
            key = ev * B + jv
            skey, _sv = plsc.sort_key_val(key, jv)
            senv = lax.shift_right_logical(skey, B.bit_length() - 1)
            sj = lax.bitwise_and(skey, B - 1)
            nxt = jnp.take(senv, jnp.minimum(lane + 1, L - 1), axis=0,
                           mode="wrap")
            m = jnp.logical_or(senv != nxt, lane == L - 1)
            plsc.store_scatter(winbuf, [senv], sj, mask=m)
            return _

        lax.fori_loop(0, ncf, win_body, None)

        for c in range(nck):
            es = env_full[pl.ds(base + c * L, L)]
            wv = plsc.load_gather(winbuf, [es])
            aidx[pl.ds(c * L, L)] = plsc.load_gather(obs_full, [wv])
            rww[pl.ds(c * L, L)] = plsc.load_gather(rew_full, [wv])

        acp = pltpu.make_async_copy(atab_hbm.at[aidx], abuf, sem_a)
        acp.start()
        for cp in dcopies:
            cp.wait()
        acp.wait()

        inv = jnp.float32(1.0 / WINDOW)

        def row_body(i, _):
            r = plsc.load_gather(rww, [jnp.full((L,), i, jnp.int32)])
            for c in range(D // L):
                sl = pl.ds(c * L, L)
                acc = abuf[i, sl] * r
                for t in range(nacc):
                    acc = acc + dbuf[t, i, sl]
                obuf[i, sl] = acc * inv
            return _

        lax.fori_loop(0, bw, row_body, None)

        pltpu.sync_copy(obuf, out_hbm.at[pl.ds(base, bw)])

    return k


def kernel(data, action_table, rew, env_id, obs_next_idx, length):
    T, B, D = data.shape
    V = action_table.shape[0]
    data_flat = data.reshape(T * B, D)
    start = jnp.int32(length) - WINDOW
    off = (start + lax.iota(jnp.int32, L)) * B
    return _make_kernel(B, D, V, T)(
        data_flat, action_table, rew, env_id, obs_next_idx, off)


# speedup vs baseline: 5.1606x; 2.5536x over previous
import functools

import jax
import jax.numpy as jnp
from jax import lax
from jax.experimental import pallas as pl
from jax.experimental.pallas import tpu as pltpu
from jax.experimental.pallas import tpu_sc as plsc

WINDOW = 10
NC = 2
NS = 16
L = 16


def _make_kernel(B, D, V, T):
    NW = NC * NS
    bw = B // NW
    nck = bw // L
    ncf = B // L
    nacc = WINDOW - 1
    mesh = plsc.VectorSubcoreMesh(core_axis_name="c", subcore_axis_name="s")

    @functools.partial(
        pl.kernel,
        mesh=mesh,
        out_type=jax.ShapeDtypeStruct((B, D), jnp.float32),
        compiler_params=pltpu.CompilerParams(
            needs_layout_passes=False, use_tc_tiling_on_sc=False),
        scratch_types=[
            pltpu.VMEM((B,), jnp.int32),
            pltpu.VMEM((B,), jnp.int32),
            pltpu.VMEM((B,), jnp.float32),
            pltpu.VMEM((B,), jnp.int32),
            pltpu.VMEM((nacc, bw), jnp.int32),
            pltpu.VMEM((bw,), jnp.int32),
            pltpu.VMEM((bw,), jnp.float32),
            pltpu.VMEM((nacc, bw, D), jnp.float32),
            pltpu.VMEM((bw, D), jnp.float32),
            pltpu.VMEM((bw, D), jnp.float32),
            pltpu.SemaphoreType.DMA,
            pltpu.SemaphoreType.DMA,
        ],
    )
    def k(data_hbm, atab_hbm, rew_hbm, env_hbm, obs_hbm, out_hbm,
          env_full, obs_full, rew_full, winbuf, idx2, aidx, rww,
          dbuf, abuf, obuf, sem_d, sem_a):
        wid = lax.axis_index("s") * NC + lax.axis_index("c")
        base = wid * bw

        pltpu.sync_copy(env_hbm, env_full)
        pltpu.sync_copy(obs_hbm, obs_full)
        pltpu.sync_copy(rew_hbm, rew_full)

        for t in range(nacc):
            for c in range(nck):
                ev = env_full[pl.ds(base + c * L, L)]
                idx2[t, pl.ds(c * L, L)] = ev + jnp.int32(t * B)

        dcopies = []
        for t in range(nacc):
            cp = pltpu.make_async_copy(data_hbm.at[idx2.at[t]], dbuf.at[t], sem_d)
            cp.start()
            dcopies.append(cp)

        lane = lax.iota(jnp.int32, L)

        def win_body(c, _):
            jv = lane + c * L
            ev = env_full[pl.ds(c * L, L)]
            key = ev * B + jv
            skey, _sv = plsc.sort_key_val(key, jv)
            senv = lax.shift_right_logical(skey, B.bit_length() - 1)
            sj = lax.bitwise_and(skey, B - 1)
            nxt = jnp.take(senv, jnp.minimum(lane + 1, L - 1), axis=0,
                           mode="wrap")
            m = jnp.logical_or(senv != nxt, lane == L - 1)
            plsc.store_scatter(winbuf, [senv], sj, mask=m)
            return _

        lax.fori_loop(0, ncf, win_body, None)

        for c in range(nck):
            es = env_full[pl.ds(base + c * L, L)]
            wv = plsc.load_gather(winbuf, [es])
            aidx[pl.ds(c * L, L)] = plsc.load_gather(obs_full, [wv])
            rww[pl.ds(c * L, L)] = plsc.load_gather(rew_full, [wv])

        acp = pltpu.make_async_copy(atab_hbm.at[aidx], abuf, sem_a)
        acp.start()
        for cp in dcopies:
            cp.wait()
        acp.wait()

        inv = jnp.float32(1.0 / WINDOW)

        def row_body(i, _):
            r = plsc.load_gather(rww, [jnp.full((L,), i, jnp.int32)])
            for c in range(D // L):
                sl = pl.ds(c * L, L)
                acc = abuf[i, sl] * r
                for t in range(nacc):
                    acc = acc + dbuf[t, i, sl]
                obuf[i, sl] = acc * inv
            return _

        lax.fori_loop(0, bw, row_body, None)

        pltpu.sync_copy(obuf, out_hbm.at[pl.ds(base, bw)])

    return k


def kernel(data, action_table, rew, env_id, obs_next_idx, length):
    T, B, D = data.shape
    V = action_table.shape[0]
    window = lax.dynamic_slice_in_dim(data, jnp.int32(length) - WINDOW,
                                      WINDOW - 1, axis=0)
    wflat = window.reshape((WINDOW - 1) * B, D)
    return _make_kernel(B, D, V, T)(
        wflat, action_table, rew, env_id, obs_next_idx)


# speedup vs baseline: 6.6841x; 1.2952x over previous
import functools

import jax
import jax.numpy as jnp
from jax import lax
from jax.experimental import pallas as pl
from jax.experimental.pallas import tpu as pltpu
from jax.experimental.pallas import tpu_sc as plsc

WINDOW = 10
NC = 2
NS = 16
L = 16


def _winner_pass(env_full, winbuf, B):
    lane = lax.iota(jnp.int32, L)
    shift = B.bit_length() - 1

    def win_body(c, carry):
        jv = lane + c * L
        ev = env_full[pl.ds(c * L, L)]
        key = ev * B + jv
        skey, _sv = plsc.sort_key_val(key, jv)
        senv = lax.shift_right_logical(skey, shift)
        sj = lax.bitwise_and(skey, B - 1)
        nxt = jnp.take(senv, jnp.minimum(lane + 1, L - 1), axis=0, mode="wrap")
        m = jnp.logical_or(senv != nxt, lane == L - 1)
        plsc.store_scatter(winbuf, [senv], sj, mask=m)
        return carry

    lax.fori_loop(0, B // L, win_body, None)


def _make_act_kernel(B, D, V):
    NW = NC * NS
    dpw = D // NW
    mesh = plsc.VectorSubcoreMesh(core_axis_name="c", subcore_axis_name="s")

    @functools.partial(
        pl.kernel,
        mesh=mesh,
        out_type=jax.ShapeDtypeStruct((D, B), jnp.float32),
        compiler_params=pltpu.CompilerParams(needs_layout_passes=False),
        scratch_types=[
            pltpu.VMEM((B,), jnp.int32),
            pltpu.VMEM((B,), jnp.int32),
            pltpu.VMEM((B,), jnp.float32),
            pltpu.VMEM((B,), jnp.int32),
            pltpu.VMEM((B,), jnp.int32),
            pltpu.VMEM((B,), jnp.float32),
            pltpu.VMEM((B,), jnp.float32),
            pltpu.VMEM((V,), jnp.float32),
            pltpu.SemaphoreType.DMA,
        ],
    )
    def ka(atab_hbm, env_hbm, obs_hbm, rew_hbm, out_hbm,
           env_full, obs_full, rew_full, winbuf, aidx, rww, orow, arow, sem):
        wid = lax.axis_index("s") * NC + lax.axis_index("c")
        d0 = wid * dpw

        cp0 = pltpu.make_async_copy(atab_hbm.at[d0], arow, sem)
        cp0.start()

        pltpu.sync_copy(env_hbm, env_full)
        pltpu.sync_copy(obs_hbm, obs_full)
        pltpu.sync_copy(rew_hbm, rew_full)

        _winner_pass(env_full, winbuf, B)

        def res_body(c, carry):
            sl = pl.ds(c * L, L)
            wv = plsc.load_gather(winbuf, [env_full[sl]])
            aidx[sl] = plsc.load_gather(obs_full, [wv])
            rww[sl] = plsc.load_gather(rew_full, [wv])
            return carry

        lax.fori_loop(0, B // L, res_body, None)

        for j in range(dpw):
            if j == 0:
                cp0.wait()
            else:
                pltpu.sync_copy(atab_hbm.at[d0 + j], arow)

            def g_body(c, carry):
                sl = pl.ds(c * L, L)
                g = plsc.load_gather(arow, [aidx[sl]])
                orow[sl] = g * rww[sl]
                return carry

            lax.fori_loop(0, B // L, g_body, None)
            pltpu.sync_copy(orow, out_hbm.at[d0 + j])

    return ka


def _make_sum_kernel(B, D):
    NW = NC * NS
    bw = B // NW
    nck = bw // L
    nacc = WINDOW - 1
    mesh = plsc.VectorSubcoreMesh(core_axis_name="c", subcore_axis_name="s")

    @functools.partial(
        pl.kernel,
        mesh=mesh,
        out_type=jax.ShapeDtypeStruct((B, D), jnp.float32),
        compiler_params=pltpu.CompilerParams(
            needs_layout_passes=False, use_tc_tiling_on_sc=False),
        scratch_types=[
            pltpu.VMEM((B,), jnp.int32),
            pltpu.VMEM((nacc, bw), jnp.int32),
            pltpu.VMEM((nacc, bw, D), jnp.float32),
            pltpu.VMEM((D, bw), jnp.float32),
            pltpu.VMEM((bw, D), jnp.float32),
            pltpu.SemaphoreType.DMA,
        ],
    )
    def ks(data_hbm, apart_hbm, env_hbm, out_hbm,
           env_full, idx2, dbuf, apart_l, obuf, sem_d):
        wid = lax.axis_index("s") * NC + lax.axis_index("c")
        base = wid * bw

        pltpu.sync_copy(env_hbm, env_full)

        for t in range(nacc):
            for c in range(nck):
                ev = env_full[pl.ds(base + c * L, L)]
                idx2[t, pl.ds(c * L, L)] = ev + jnp.int32(t * B)

        dcopies = []
        for t in range(nacc):
            cp = pltpu.make_async_copy(data_hbm.at[idx2.at[t]], dbuf.at[t],
                                       sem_d)
            cp.start()
            dcopies.append(cp)

        pltpu.sync_copy(apart_hbm.at[:, pl.ds(base, bw)], apart_l)
        for cp in dcopies:
            cp.wait()

        inv = jnp.float32(1.0 / WINDOW)
        lane = lax.iota(jnp.int32, L)

        def row_body(i, carry):
            for c in range(D // L):
                sl = pl.ds(c * L, L)
                acc = plsc.load_gather(
                    apart_l, [lane + c * L, jnp.full((L,), i, jnp.int32)])
                for t in range(nacc):
                    acc = acc + dbuf[t, i, sl]
                obuf[i, sl] = acc * inv
            return carry

        lax.fori_loop(0, bw, row_body, None)

        pltpu.sync_copy(obuf, out_hbm.at[pl.ds(base, bw)])

    return ks


def kernel(data, action_table, rew, env_id, obs_next_idx, length):
    T, B, D = data.shape
    V = action_table.shape[0]
    atab_t = action_table.T
    apart = _make_act_kernel(B, D, V)(atab_t, env_id, obs_next_idx, rew)
    window = lax.dynamic_slice_in_dim(data, jnp.int32(length) - WINDOW,
                                      WINDOW - 1, axis=0)
    wflat = window.reshape((WINDOW - 1) * B, D)
    return _make_sum_kernel(B, D)(wflat, apart, env_id)


# speedup vs baseline: 10.3149x; 1.5432x over previous
import functools

import jax
import jax.numpy as jnp
from jax import lax
from jax.experimental import pallas as pl
from jax.experimental.pallas import tpu as pltpu
from jax.experimental.pallas import tpu_sc as plsc

WINDOW = 10
NC = 2
NS = 16
L = 16


def _make_kernel(T, B, D, V):
    NW = NC * NS
    dpw = D // NW
    nacc = WINDOW - 1
    ncf = B // L
    mesh = plsc.VectorSubcoreMesh(core_axis_name="c", subcore_axis_name="s")

    @functools.partial(
        pl.kernel,
        mesh=mesh,
        out_type=jax.ShapeDtypeStruct((D, B), jnp.float32),
        compiler_params=pltpu.CompilerParams(needs_layout_passes=False),
        scratch_types=[
            pltpu.VMEM((B,), jnp.int32),
            pltpu.VMEM((B,), jnp.int32),
            pltpu.VMEM((B,), jnp.float32),
            pltpu.VMEM((B,), jnp.float32),
            pltpu.VMEM((L,), jnp.int32),
            pltpu.VMEM((dpw, L), jnp.int32),
            pltpu.SemaphoreType.DMA,
        ],
    )
    def k(data_hbm, atab_hbm, rew_hbm, env_hbm, obs_hbm, toff_hbm, out_hbm,
          env_full, aidx, rww, orow, toff_v, ridx, sem_d):
        wid = lax.axis_index("s") * NC + lax.axis_index("c")
        d0 = wid * dpw
        lane = lax.iota(jnp.int32, L)

        pltpu.sync_copy(env_hbm, env_full)
        pltpu.sync_copy(toff_hbm, toff_v)

        tv = toff_v[...]
        tvc = jnp.take(tv, jnp.minimum(lane, nacc - 1), axis=0, mode="wrap")
        for j in range(dpw):
            ridx[j, :] = tvc + jnp.int32(d0 + j)

        def phase1(obs_full, rew_full, winbuf):
            pltpu.sync_copy(obs_hbm, obs_full)
            pltpu.sync_copy(rew_hbm, rew_full)
            shift = B.bit_length() - 1

            def win_body(c, carry):
                jv = lane + c * L
                ev = env_full[pl.ds(c * L, L)]
                key = ev * B + jv
                skey, _sv = plsc.sort_key_val(key, jv)
                senv = lax.shift_right_logical(skey, shift)
                sj = lax.bitwise_and(skey, B - 1)
                nxt = jnp.take(senv, jnp.minimum(lane + 1, L - 1), axis=0,
                               mode="wrap")
                m = jnp.logical_or(senv != nxt, lane == L - 1)
                plsc.store_scatter(winbuf, [senv], sj, mask=m)
                return carry

            lax.fori_loop(0, ncf, win_body, None)

            def res_body(c, carry):
                sl = pl.ds(c * L, L)
                wv = plsc.load_gather(winbuf, [env_full[sl]])
                aidx[sl] = plsc.load_gather(obs_full, [wv])
                rww[sl] = plsc.load_gather(rew_full, [wv])
                return carry

            lax.fori_loop(0, ncf, res_body, None)

        pl.run_scoped(phase1,
                      pltpu.VMEM((B,), jnp.int32),
                      pltpu.VMEM((B,), jnp.float32),
                      pltpu.VMEM((B,), jnp.int32))

        inv = jnp.float32(1.0 / WINDOW)

        for j in range(dpw):
            def phase2(arow):
                pltpu.sync_copy(atab_hbm.at[d0 + j], arow)

                def a_body(c, carry):
                    sl = pl.ds(c * L, L)
                    g = plsc.load_gather(arow, [aidx[sl]])
                    orow[sl] = g * rww[sl]
                    return carry

                lax.fori_loop(0, ncf, a_body, None)

            pl.run_scoped(phase2, pltpu.VMEM((V,), jnp.float32))

            def phase3(dstage, wsum):
                pltpu.async_copy(data_hbm.at[ridx.at[j]], dstage, sem_d).wait()

                def s_body(c, carry):
                    sl = pl.ds(c * L, L)
                    acc = dstage[0, sl]
                    for t in range(1, nacc):
                        acc = acc + dstage[t, sl]
                    wsum[sl] = acc
                    return carry

                lax.fori_loop(0, ncf, s_body, None)

                def o_body(c, carry):
                    sl = pl.ds(c * L, L)
                    g = plsc.load_gather(wsum, [env_full[sl]])
                    orow[sl] = (g + orow[sl]) * inv
                    return carry

                lax.fori_loop(0, ncf, o_body, None)

            pl.run_scoped(phase3,
                          pltpu.VMEM((L, B), jnp.float32),
                          pltpu.VMEM((B,), jnp.float32))
            pltpu.sync_copy(orow, out_hbm.at[d0 + j])

    return k


def kernel(data, action_table, rew, env_id, obs_next_idx, length):
    T, B, D = data.shape
    V = action_table.shape[0]
    data_t = jnp.transpose(data, (0, 2, 1)).reshape(T * D, B)
    atab_t = action_table.T
    start = jnp.int32(length) - WINDOW
    toff = (start + lax.iota(jnp.int32, L)) * D
    out_t = _make_kernel(T, B, D, V)(
        data_t, atab_t, rew, env_id, obs_next_idx, toff)
    return out_t.T


# speedup vs baseline: 10.6890x; 1.0363x over previous
import functools

import jax
import jax.numpy as jnp
from jax import lax
from jax.experimental import pallas as pl
from jax.experimental.pallas import tpu as pltpu
from jax.experimental.pallas import tpu_sc as plsc

WINDOW = 10
NC = 2
NS = 16
L = 16


def _make_kernel(T, B, D, V):
    NW = NC * NS
    dpw = D // NW
    nacc = WINDOW - 1
    ncf = B // L
    mesh = plsc.VectorSubcoreMesh(core_axis_name="c", subcore_axis_name="s")

    @functools.partial(
        pl.kernel,
        mesh=mesh,
        out_type=jax.ShapeDtypeStruct((D, B), jnp.float32),
        compiler_params=pltpu.CompilerParams(needs_layout_passes=False),
        scratch_types=[
            pltpu.VMEM((B,), jnp.int32),
            pltpu.VMEM((B,), jnp.int32),
            pltpu.VMEM((B,), jnp.float32),
            pltpu.VMEM((B,), jnp.float32),
            pltpu.VMEM((L,), jnp.int32),
            pltpu.VMEM((dpw, L), jnp.int32),
            pltpu.SemaphoreType.DMA,
            pltpu.SemaphoreType.DMA,
            pltpu.SemaphoreType.DMA,
        ],
    )
    def k(data_hbm, atab_hbm, rew_hbm, env_hbm, obs_hbm, toff_hbm, out_hbm,
          env_full, aidx, rww, orow, toff_v, ridx, sem_t, sem_g, sem_o):
        wid = lax.axis_index("s") * NC + lax.axis_index("c")
        d0 = wid * dpw
        lane = lax.iota(jnp.int32, L)

        pltpu.sync_copy(env_hbm, env_full)
        pltpu.sync_copy(toff_hbm, toff_v)

        tv = toff_v[...]
        tvc = jnp.take(tv, jnp.minimum(lane, nacc - 1), axis=0, mode="wrap")
        for j in range(dpw):
            ridx[j, :] = tvc + jnp.int32(d0 + j)

        def phase1(obs_full, rew_full, winbuf):
            pltpu.sync_copy(obs_hbm, obs_full)
            pltpu.sync_copy(rew_hbm, rew_full)
            shift = B.bit_length() - 1

            def win_body(c, carry):
                jv = lane + c * L
                ev = env_full[pl.ds(c * L, L)]
                key = ev * B + jv
                skey, _sv = plsc.sort_key_val(key, jv)
                senv = lax.shift_right_logical(skey, shift)
                sj = lax.bitwise_and(skey, B - 1)
                nxt = jnp.take(senv, jnp.minimum(lane + 1, L - 1), axis=0,
                               mode="wrap")
                m = jnp.logical_or(senv != nxt, lane == L - 1)
                plsc.store_scatter(winbuf, [senv], sj, mask=m)
                return carry

            lax.fori_loop(0, ncf, win_body, None)

            def res_body(c, carry):
                sl = pl.ds(c * L, L)
                wv = plsc.load_gather(winbuf, [env_full[sl]])
                aidx[sl] = plsc.load_gather(obs_full, [wv])
                rww[sl] = plsc.load_gather(rew_full, [wv])
                return carry

            lax.fori_loop(0, ncf, res_body, None)

        inv = jnp.float32(1.0 / WINDOW)
        out_cps = []

        for j in range(dpw):
            def phase2(arow):
                cp = pltpu.make_async_copy(atab_hbm.at[d0 + j], arow, sem_t)
                cp.start()
                if j == 0:
                    pl.run_scoped(phase1,
                                  pltpu.VMEM((B,), jnp.int32),
                                  pltpu.VMEM((B,), jnp.float32),
                                  pltpu.VMEM((B,), jnp.int32))
                else:
                    out_cps[-1].wait()
                cp.wait()

                def a_body(c, carry):
                    sl = pl.ds(c * L, L)
                    g = plsc.load_gather(arow, [aidx[sl]])
                    orow[sl] = g * rww[sl]
                    return carry

                lax.fori_loop(0, ncf, a_body, None)

            pl.run_scoped(phase2, pltpu.VMEM((V,), jnp.float32))

            def phase3(dstage, wsum):
                pltpu.async_copy(data_hbm.at[ridx.at[j]], dstage, sem_g).wait()

                def s_body(c, carry):
                    sl = pl.ds(c * L, L)
                    acc = dstage[0, sl]
                    for t in range(1, nacc):
                        acc = acc + dstage[t, sl]
                    wsum[sl] = acc
                    return carry

                lax.fori_loop(0, ncf, s_body, None)

                def o_body(c, carry):
                    sl = pl.ds(c * L, L)
                    g = plsc.load_gather(wsum, [env_full[sl]])
                    orow[sl] = (g + orow[sl]) * inv
                    return carry

                lax.fori_loop(0, ncf, o_body, None)

            pl.run_scoped(phase3,
                          pltpu.VMEM((L, B), jnp.float32),
                          pltpu.VMEM((B,), jnp.float32))
            ocp = pltpu.make_async_copy(orow, out_hbm.at[d0 + j], sem_o)
            ocp.start()
            out_cps.append(ocp)

        out_cps[-1].wait()

    return k


def kernel(data, action_table, rew, env_id, obs_next_idx, length):
    T, B, D = data.shape
    V = action_table.shape[0]
    data_t = jnp.transpose(data, (0, 2, 1)).reshape(T * D, B)
    atab_t = action_table.T
    start = jnp.int32(length) - WINDOW
    toff = (start + lax.iota(jnp.int32, L)) * D
    out_t = _make_kernel(T, B, D, V)(
        data_t, atab_t, rew, env_id, obs_next_idx, toff)
    return out_t.T
